# baseline (speedup 1.0000x reference)
import functools

import jax
import jax.numpy as jnp
from jax import lax
from jax.experimental import pallas as pl
from jax.experimental.pallas import tpu as pltpu
from jax.experimental.pallas import tpu_sc as plsc

N_VOCAB = 100000
D_MODEL = 128
N_SEQ = 16384
SEQ_LEN = 50
NUM_WORKERS = 32
NCALLS = 4
SEQ_PER_CALL = N_SEQ // NCALLS
SEQ_PER_WORKER = SEQ_PER_CALL // NUM_WORKERS
NBUF = 4
LOOKAHEAD = NBUF - 1


def _gather_kernel(idx_hbm, table_hbm, out_hbm,
                   idx_v, b0, b1, b2, b3, g0, g1, g2, g3, s0, s1, s2, s3):
    wid = lax.axis_index("s") * 2 + lax.axis_index("c")
    base = wid * SEQ_PER_WORKER
    pltpu.sync_copy(idx_hbm.at[pl.ds(base, SEQ_PER_WORKER)], idx_v)

    bufs = (b0, b1, b2, b3)
    gsems = (g0, g1, g2, g3)
    ssems = (s0, s1, s2, s3)

    def gather_desc(seq, slot):
        return pltpu.make_async_copy(table_hbm.at[idx_v.at[seq]],
                                     bufs[slot], gsems[slot])

    def store_desc(seq, slot):
        return pltpu.make_async_copy(bufs[slot], out_hbm.at[base + seq],
                                     ssems[slot])

    for c in range(LOOKAHEAD):
        gather_desc(c, c).start()

    def body(o, carry):
        for b in range(NBUF):
            t = NBUF * o + b
            gather_desc(t, b).wait()
            store_desc(t, b).start()
            f = t + LOOKAHEAD
            fslot = (b + LOOKAHEAD) % NBUF

            @pl.when(jnp.logical_and(f < SEQ_PER_WORKER, t >= 1))
            def _():
                store_desc(t - 1, fslot).wait()

            @pl.when(f < SEQ_PER_WORKER)
            def _():
                gather_desc(f, fslot).start()
        return carry

    lax.fori_loop(0, SEQ_PER_WORKER // NBUF, body, 0)

    for c in range(SEQ_PER_WORKER - NBUF, SEQ_PER_WORKER):
        store_desc(c, c % NBUF).wait()


def kernel(inputs, embedding_weight):
    idx = inputs.astype(jnp.int32)
    mesh = plsc.VectorSubcoreMesh(core_axis_name="c", subcore_axis_name="s")
    run = functools.partial(
        pl.kernel,
        mesh=mesh,
        out_type=jax.ShapeDtypeStruct((SEQ_PER_CALL, SEQ_LEN, D_MODEL),
                                      jnp.float32),
        scratch_types=(
            [pltpu.VMEM((SEQ_PER_WORKER, SEQ_LEN), jnp.int32)]
            + [pltpu.VMEM((SEQ_LEN, D_MODEL), jnp.float32)] * NBUF
            + [pltpu.SemaphoreType.DMA] * (2 * NBUF)
        ),
    )(_gather_kernel)
    parts = [run(idx[k * SEQ_PER_CALL:(k + 1) * SEQ_PER_CALL], embedding_weight)
             for k in range(NCALLS)]
    return jnp.concatenate(parts, axis=0)


# speedup vs baseline: 1.7650x; 1.7650x over previous
import functools

import jax
import jax.numpy as jnp
from jax import lax
from jax.experimental import pallas as pl
from jax.experimental.pallas import tpu as pltpu
from jax.experimental.pallas import tpu_sc as plsc

N_VOCAB = 100000
D_MODEL = 128
N_SEQ = 16384
SEQ_LEN = 50
NUM_WORKERS = 32
SEQ_PER_WORKER = N_SEQ // NUM_WORKERS
NBUF = 4
LOOKAHEAD = NBUF - 1


def _gather_kernel(idx_hbm, table_hbm, out_hbm,
                   idx_v, b0, b1, b2, b3, g0, g1, g2, g3, s0, s1, s2, s3):
    wid = lax.axis_index("s") * 2 + lax.axis_index("c")
    base = wid * SEQ_PER_WORKER
    pltpu.sync_copy(idx_hbm.at[pl.ds(base, SEQ_PER_WORKER)], idx_v)

    bufs = (b0, b1, b2, b3)
    gsems = (g0, g1, g2, g3)
    ssems = (s0, s1, s2, s3)

    def gather_desc(seq, slot):
        return pltpu.make_async_copy(table_hbm.at[idx_v.at[seq]],
                                     bufs[slot], gsems[slot])

    def store_desc(seq, slot):
        return pltpu.make_async_copy(bufs[slot], out_hbm.at[base + seq],
                                     ssems[slot])

    for c in range(LOOKAHEAD):
        gather_desc(c, c).start()

    def body(o, carry):
        for b in range(NBUF):
            t = NBUF * o + b
            gather_desc(t, b).wait()
            store_desc(t, b).start()
            f = t + LOOKAHEAD
            fslot = (b + LOOKAHEAD) % NBUF

            @pl.when(jnp.logical_and(f < SEQ_PER_WORKER, t >= 1))
            def _():
                store_desc(t - 1, fslot).wait()

            @pl.when(f < SEQ_PER_WORKER)
            def _():
                gather_desc(f, fslot).start()
        return carry

    lax.fori_loop(0, SEQ_PER_WORKER // NBUF, body, 0)

    for c in range(SEQ_PER_WORKER - NBUF, SEQ_PER_WORKER):
        store_desc(c, c % NBUF).wait()


def kernel(inputs, embedding_weight):
    idx = inputs.astype(jnp.int32)
    mesh = plsc.VectorSubcoreMesh(core_axis_name="c", subcore_axis_name="s")
    run = functools.partial(
        pl.kernel,
        mesh=mesh,
        out_type=jax.ShapeDtypeStruct((N_SEQ, SEQ_LEN, D_MODEL), jnp.float32),
        compiler_params=pltpu.CompilerParams(use_tc_tiling_on_sc=True),
        scratch_types=(
            [pltpu.VMEM((SEQ_PER_WORKER, SEQ_LEN), jnp.int32)]
            + [pltpu.VMEM((SEQ_LEN, D_MODEL), jnp.float32)] * NBUF
            + [pltpu.SemaphoreType.DMA] * (2 * NBUF)
        ),
    )(_gather_kernel)
    return run(idx, embedding_weight)


# speedup vs baseline: 1.7806x; 1.0088x over previous
import functools

import jax
import jax.numpy as jnp
from jax import lax
from jax.experimental import pallas as pl
from jax.experimental.pallas import tpu as pltpu
from jax.experimental.pallas import tpu_sc as plsc

N_VOCAB = 100000
D_MODEL = 128
N_SEQ = 16384
SEQ_LEN = 50
NUM_WORKERS = 32
SEQ_PER_WORKER = N_SEQ // NUM_WORKERS
NBUF = 8
LOOKAHEAD = NBUF - 1


def _gather_kernel(idx_hbm, table_hbm, out_hbm,
                   idx_v, b0, b1, b2, b3, b4, b5, b6, b7,
                   g0, g1, g2, g3, g4, g5, g6, g7,
                   s0, s1, s2, s3, s4, s5, s6, s7):
    wid = lax.axis_index("s") * 2 + lax.axis_index("c")
    base = wid * SEQ_PER_WORKER
    pltpu.sync_copy(idx_hbm.at[pl.ds(base, SEQ_PER_WORKER)], idx_v)

    bufs = (b0, b1, b2, b3, b4, b5, b6, b7)
    gsems = (g0, g1, g2, g3, g4, g5, g6, g7)
    ssems = (s0, s1, s2, s3, s4, s5, s6, s7)

    def gather_desc(seq, slot):
        return pltpu.make_async_copy(table_hbm.at[idx_v.at[seq]],
                                     bufs[slot], gsems[slot])

    def store_desc(seq, slot):
        return pltpu.make_async_copy(bufs[slot], out_hbm.at[base + seq],
                                     ssems[slot])

    for c in range(LOOKAHEAD):
        gather_desc(c, c).start()

    def body(o, carry):
        for b in range(NBUF):
            t = NBUF * o + b
            gather_desc(t, b).wait()
            store_desc(t, b).start()
            f = t + LOOKAHEAD
            fslot = (b + LOOKAHEAD) % NBUF

            @pl.when(jnp.logical_and(f < SEQ_PER_WORKER, t >= 1))
            def _():
                store_desc(t - 1, fslot).wait()

            @pl.when(f < SEQ_PER_WORKER)
            def _():
                gather_desc(f, fslot).start()
        return carry

    lax.fori_loop(0, SEQ_PER_WORKER // NBUF, body, 0)

    for c in range(SEQ_PER_WORKER - NBUF, SEQ_PER_WORKER):
        store_desc(c, c % NBUF).wait()


def kernel(inputs, embedding_weight):
    idx = inputs.astype(jnp.int32)
    mesh = plsc.VectorSubcoreMesh(core_axis_name="c", subcore_axis_name="s")
    run = functools.partial(
        pl.kernel,
        mesh=mesh,
        out_type=jax.ShapeDtypeStruct((N_SEQ, SEQ_LEN, D_MODEL), jnp.float32),
        scratch_types=(
            [pltpu.VMEM((SEQ_PER_WORKER, SEQ_LEN), jnp.int32)]
            + [pltpu.VMEM((SEQ_LEN, D_MODEL), jnp.float32)] * NBUF
            + [pltpu.SemaphoreType.DMA] * (2 * NBUF)
        ),
    )(_gather_kernel)
    return run(idx, embedding_weight)
